# baseline (speedup 1.0000x reference)
import functools
import math

import jax
import jax.numpy as jnp
from jax import lax
from jax.experimental import pallas as pl
from jax.experimental.pallas import tpu as pltpu
from jax.experimental.pallas import tpu_sc as plsc

LN2 = math.log(2.0)
F32 = jnp.float32


def _sp2(x):
    return jnp.log(0.5 + 0.5 * jnp.exp(x))


def _dot(a, b):
    return jnp.dot(a, b, preferred_element_type=F32)


def _state_pre(s8, Wsf1, bsf1, Wsf2, bsf2, W1_s, b1, Wn_s, bn1):
    def body(s_ref, w1_ref, b1_ref, w2_ref, b2_ref, we_ref, be_ref, wn_ref,
             bn_ref, s1_ref, ce_ref, cn_ref):
        t = _sp2(_dot(s_ref[...], w1_ref[...]) + b1_ref[...])
        s1 = _sp2(_dot(t, w2_ref[...]) + b2_ref[...])
        s1_ref[...] = s1
        ce_ref[...] = _dot(s1, we_ref[...]) + be_ref[...]
        cn_ref[...] = _dot(s1, wn_ref[...]) + bn_ref[...]

    return pl.pallas_call(
        body,
        out_shape=(
            jax.ShapeDtypeStruct((8, 128), F32),
            jax.ShapeDtypeStruct((8, 256), F32),
            jax.ShapeDtypeStruct((8, 256), F32),
        ),
    )(s8, Wsf1, bsf1, Wsf2, bsf2, W1_s, b1, Wn_s, bn1)


def _node_pre(h, Wnf1, bnf1, Wnf2, bnf2, bn):
    n = h.shape[0]
    grid = n // bn

    def body(h_ref, w1_ref, b1_ref, w2_ref, b2_ref, h1_ref):
        t = _sp2(_dot(h_ref[...], w1_ref[...]) + b1_ref[...])
        h1_ref[...] = _sp2(_dot(t, w2_ref[...]) + b2_ref[...])

    wspec = lambda shape: pl.BlockSpec(shape, lambda i: (0, 0))
    return pl.pallas_call(
        body,
        grid=(grid,),
        in_specs=[
            pl.BlockSpec((bn, 128), lambda i: (i, 0)),
            wspec((128, 256)), wspec((1, 256)),
            wspec((256, 128)), wspec((1, 128)),
        ],
        out_specs=pl.BlockSpec((bn, 128), lambda i: (i, 0)),
        out_shape=jax.ShapeDtypeStruct((n, 128), F32),
    )(h, Wnf1, bnf1, Wnf2, bnf2)


def _edge_stage(e, hs, hd, Wef1, bef1, Wef2, bef2, W1_hs, W1_hd, W1_e, ce,
                W2, b2, W3, b3, be, eoff):
    cnt = hs.shape[0]
    grid = cnt // be
    ob = eoff // be

    def body(e_ref, hs_ref, hd_ref, wef1_ref, bef1_ref, wef2_ref, bef2_ref,
             whs_ref, whd_ref, we_ref, ce_ref, w2_ref, b2_ref, w3_ref, b3_ref,
             e2_ref, acc_ref):
        t = _sp2(_dot(e_ref[...], wef1_ref[...]) + bef1_ref[...])
        e1 = _sp2(_dot(t, wef2_ref[...]) + bef2_ref[...])
        x = (_dot(hs_ref[...], whs_ref[...]) + _dot(hd_ref[...], whd_ref[...])
             + _dot(e1, we_ref[...]) + ce_ref[0:1, :])
        x = _sp2(x)
        x = _sp2(_dot(x, w2_ref[...]) + b2_ref[...])
        e2 = _sp2(_dot(x, w3_ref[...]) + b3_ref[...])
        e2_ref[...] = e2
        part = jnp.sum(e2.reshape(be // 8, 8, 128), axis=0)

        @pl.when(pl.program_id(0) == 0)
        def _():
            acc_ref[...] = jnp.zeros_like(acc_ref)

        acc_ref[...] += part

    wspec = lambda shape: pl.BlockSpec(shape, lambda i: (0, 0))
    espec = lambda w: pl.BlockSpec((be, w), lambda i: (i, 0))
    return pl.pallas_call(
        body,
        grid=(grid,),
        in_specs=[
            pl.BlockSpec((be, 128), lambda i: (i + ob, 0)),
            espec(128), espec(128),
            wspec((128, 256)), wspec((1, 256)), wspec((256, 128)),
            wspec((1, 128)),
            wspec((128, 256)), wspec((128, 256)), wspec((128, 256)),
            wspec((8, 256)),
            wspec((256, 256)), wspec((1, 256)), wspec((256, 128)),
            wspec((1, 128)),
        ],
        out_specs=(
            pl.BlockSpec((be, 128), lambda i: (i, 0)),
            pl.BlockSpec((8, 128), lambda i: (0, 0)),
        ),
        out_shape=(
            jax.ShapeDtypeStruct((cnt, 128), F32),
            jax.ShapeDtypeStruct((8, 128), F32),
        ),
    )(e, hs, hd, Wef1, bef1, Wef2, bef2, W1_hs, W1_hd, W1_e, ce, W2, b2,
      W3, b3)


def _node_stage(hhp0, hhp1, degp, h1, Wn_hh, Wn_h, cn, Wn2, bn2, Wn3, bn3,
                bn):
    n = h1.shape[0]
    grid = n // bn

    def body(hhp0_ref, hhp1_ref, degp_ref, h1_ref, whh_ref, wh_ref, cn_ref,
             w2_ref, b2_ref, w3_ref, b3_ref, h2_ref, acc_ref):
        hh_sum = (hhp0_ref[0] + hhp0_ref[1]) + (hhp1_ref[0] + hhp1_ref[1])
        deg = degp_ref[0, :, 0:1] + degp_ref[1, :, 0:1]
        hh = hh_sum / jnp.maximum(deg, 1.0)
        x = (_dot(hh, whh_ref[...]) + _dot(h1_ref[...], wh_ref[...])
             + cn_ref[0:1, :])
        x = _sp2(x)
        x = _sp2(_dot(x, w2_ref[...]) + b2_ref[...])
        h2 = _sp2(_dot(x, w3_ref[...]) + b3_ref[...])
        h2_ref[...] = h2
        part = jnp.sum(h2.reshape(bn // 8, 8, 128), axis=0)

        @pl.when(pl.program_id(0) == 0)
        def _():
            acc_ref[...] = jnp.zeros_like(acc_ref)

        acc_ref[...] += part

    wspec = lambda shape: pl.BlockSpec(shape, lambda i: (0, 0))
    return pl.pallas_call(
        body,
        grid=(grid,),
        in_specs=[
            pl.BlockSpec((2, bn, 128), lambda i: (0, i, 0)),
            pl.BlockSpec((2, bn, 128), lambda i: (0, i, 0)),
            pl.BlockSpec((2, bn, 128), lambda i: (0, i, 0)),
            pl.BlockSpec((bn, 128), lambda i: (i, 0)),
            wspec((128, 256)), wspec((128, 256)), wspec((8, 256)),
            wspec((256, 256)), wspec((1, 256)), wspec((256, 128)),
            wspec((1, 128)),
        ],
        out_specs=(
            pl.BlockSpec((bn, 128), lambda i: (i, 0)),
            pl.BlockSpec((8, 128), lambda i: (0, 0)),
        ),
        out_shape=(
            jax.ShapeDtypeStruct((n, 128), F32),
            jax.ShapeDtypeStruct((8, 128), F32),
        ),
    )(hhp0, hhp1, degp, h1, Wn_hh, Wn_h, cn, Wn2, bn2, Wn3, bn3)


def _state_stage(s1_8, sum_e2, sum_h2, n_edges, n_nodes, Ws_s1, Ws_ue, Ws_un,
                 bs1, Ws2, bs2, Ws3, bs3):
    inv_e = 1.0 / float(n_edges)
    inv_n = 1.0 / float(n_nodes)

    def body(s1_ref, se_ref, sh_ref, wa_ref, wb_ref, wc_ref, b1_ref, w2_ref,
             b2_ref, w3_ref, b3_ref, out_ref):
        ue = jnp.sum(se_ref[...], axis=0, keepdims=True) * inv_e
        un = jnp.sum(sh_ref[...], axis=0, keepdims=True) * inv_n
        x = (_dot(s1_ref[0:1, :], wa_ref[...]) + _dot(ue, wb_ref[...])
             + _dot(un, wc_ref[...]) + b1_ref[...])
        x = _sp2(x)
        x = _sp2(_dot(x, w2_ref[...]) + b2_ref[...])
        out_ref[...] = _sp2(_dot(x, w3_ref[...]) + b3_ref[...])

    return pl.pallas_call(
        body,
        out_shape=jax.ShapeDtypeStruct((1, 128), F32),
    )(s1_8, sum_e2, sum_h2, Ws_s1, Ws_ue, Ws_un, bs1, Ws2, bs2, Ws3, bs3)


def _sc_gather(h1, src, dst, goff, cnt):
    NW = 32
    EPW = cnt // NW
    K = 40
    SUB = 5
    KB = K * SUB
    chunks = EPW // KB
    mesh = plsc.VectorSubcoreMesh(core_axis_name="c", subcore_axis_name="s")

    @functools.partial(
        pl.kernel,
        out_type=(
            jax.ShapeDtypeStruct((cnt, 128), F32),
            jax.ShapeDtypeStruct((cnt, 128), F32),
        ),
        mesh=mesh,
        scratch_types=[
            pltpu.VMEM((2, SUB, K), jnp.int32),
            pltpu.VMEM((2, SUB, K), jnp.int32),
            pltpu.VMEM((2, KB, 128), F32),
            pltpu.VMEM((2, KB, 128), F32),
            pltpu.SemaphoreType.DMA,
            pltpu.SemaphoreType.DMA,
            pltpu.SemaphoreType.DMA,
            pltpu.SemaphoreType.DMA,
            pltpu.SemaphoreType.DMA,
            pltpu.SemaphoreType.DMA,
        ],
    )
    def k(h1_hbm, src_hbm, dst_hbm, hs_hbm, hd_hbm,
          si_v, di_v, a_v, b_v, sem_i0, sem_i1, sem_g0, sem_g1, sem_w0,
          sem_w1):
        cid = lax.axis_index("c")
        sid = lax.axis_index("s")
        wid = sid * 2 + cid
        base = wid * EPW
        sems = ((sem_i0, sem_g0, sem_w0), (sem_i1, sem_g1, sem_w1))

        def sub(c, p):
            si, di, av, bv = si_v.at[p], di_v.at[p], a_v.at[p], b_v.at[p]
            sem_i, sem_g, sem_w = sems[p]
            off = base + c * KB
            gof = goff + off
            for j in range(SUB):
                pltpu.async_copy(src_hbm.at[pl.ds(gof + j * K, K)],
                                 si.at[j], sem_i)
                pltpu.async_copy(dst_hbm.at[pl.ds(gof + j * K, K)],
                                 di.at[j], sem_i)

            @pl.when(c > 1)
            def _():
                pltpu.make_async_copy(
                    av, hs_hbm.at[pl.ds(off - 2 * KB, KB)], sem_w).wait()
                pltpu.make_async_copy(
                    bv, hd_hbm.at[pl.ds(off - 2 * KB, KB)], sem_w).wait()

            for j in range(SUB):
                pltpu.make_async_copy(src_hbm.at[pl.ds(gof + j * K, K)],
                                      si.at[j], sem_i).wait()
                pltpu.make_async_copy(dst_hbm.at[pl.ds(gof + j * K, K)],
                                      di.at[j], sem_i).wait()
            for j in range(SUB):
                pltpu.async_copy(h1_hbm.at[si.at[j]],
                                 av.at[pl.ds(j * K, K)], sem_g)
                pltpu.async_copy(h1_hbm.at[di.at[j]],
                                 bv.at[pl.ds(j * K, K)], sem_g)
            for j in range(SUB):
                pltpu.make_async_copy(h1_hbm.at[si.at[j]],
                                      av.at[pl.ds(j * K, K)], sem_g).wait()
                pltpu.make_async_copy(h1_hbm.at[di.at[j]],
                                      bv.at[pl.ds(j * K, K)], sem_g).wait()
            pltpu.async_copy(av, hs_hbm.at[pl.ds(off, KB)], sem_w)
            pltpu.async_copy(bv, hd_hbm.at[pl.ds(off, KB)], sem_w)

        def step(c, _):
            @pl.when(c % 2 == 0)
            def _():
                sub(c, 0)

            @pl.when(c % 2 == 1)
            def _():
                sub(c, 1)

            return ()

        lax.fori_loop(0, chunks, step, ())
        for cl in (chunks - 2, chunks - 1):
            p = cl % 2
            off = base + cl * KB
            pltpu.make_async_copy(a_v.at[p], hs_hbm.at[pl.ds(off, KB)],
                                  sems[p][2]).wait()
            pltpu.make_async_copy(b_v.at[p], hd_hbm.at[pl.ds(off, KB)],
                                  sems[p][2]).wait()

    return k(h1, src, dst)


def _sc_deg(dst, ones80, zeros128):
    E = dst.shape[0]
    N = zeros128.shape[0]
    NW = 32
    NS = 16
    EPW = E // NW
    K = 80
    SUB = 5
    KB = K * SUB
    chunks = EPW // KB
    RPT = 624
    REM = N - NS * RPT
    mesh = plsc.VectorSubcoreMesh(core_axis_name="c", subcore_axis_name="s")

    @functools.partial(
        pl.kernel,
        out_type=jax.ShapeDtypeStruct((2, N, 128), F32),
        mesh=mesh,
        scratch_types=[
            pltpu.VMEM((SUB, K), jnp.int32),
            pltpu.VMEM((K, 128), F32),
            pltpu.VMEM_SHARED((N, 128), F32),
            pltpu.SemaphoreType.DMA,
        ],
    )
    def k(dst_hbm, ones_hbm, z_hbm, deg_hbm, di_v, ones_v, deg_sh, sem_i):
        cid = lax.axis_index("c")
        sid = lax.axis_index("s")
        wid = sid * 2 + cid
        base = wid * EPW

        r0 = sid * RPT
        pltpu.sync_copy(z_hbm.at[pl.ds(r0, RPT)], deg_sh.at[pl.ds(r0, RPT)])

        @pl.when(sid == NS - 1)
        def _():
            pltpu.sync_copy(z_hbm.at[pl.ds(NS * RPT, REM)],
                            deg_sh.at[pl.ds(NS * RPT, REM)])

        pltpu.sync_copy(ones_hbm, ones_v)
        plsc.subcore_barrier()

        def step(c, _):
            off = base + c * KB
            for j in range(SUB):
                pltpu.async_copy(dst_hbm.at[pl.ds(off + j * K, K)],
                                 di_v.at[j], sem_i)
            for j in range(SUB):
                pltpu.make_async_copy(dst_hbm.at[pl.ds(off + j * K, K)],
                                      di_v.at[j], sem_i).wait()
            for j in range(SUB):
                pltpu.sync_copy(ones_v, deg_sh.at[di_v.at[j]], add=True)
            return ()

        lax.fori_loop(0, chunks, step, ())
        plsc.subcore_barrier()

        pltpu.sync_copy(deg_sh.at[pl.ds(r0, RPT)],
                        deg_hbm.at[cid, pl.ds(r0, RPT)])

        @pl.when(sid == NS - 1)
        def _():
            pltpu.sync_copy(deg_sh.at[pl.ds(NS * RPT, REM)],
                            deg_hbm.at[cid, pl.ds(NS * RPT, REM)])

    return k(dst, ones80, zeros128)


def _sc_scatter(e2, dst, goff, zeros128):
    cnt = e2.shape[0]
    N = zeros128.shape[0]
    NW = 32
    NS = 16
    EPW = cnt // NW
    K = 40
    RPT = 624
    REM = N - NS * RPT
    mesh = plsc.VectorSubcoreMesh(core_axis_name="c", subcore_axis_name="s")

    SUB = 1
    KB = K * SUB
    chunks = EPW // KB

    @functools.partial(
        pl.kernel,
        out_type=jax.ShapeDtypeStruct((2, N, 128), F32),
        mesh=mesh,
        scratch_types=[
            pltpu.VMEM((2, SUB, K), jnp.int32),
            pltpu.VMEM((2, KB, 128), F32),
            pltpu.VMEM_SHARED((N, 128), F32),
            pltpu.SemaphoreType.DMA,
            pltpu.SemaphoreType.DMA,
            pltpu.SemaphoreType.DMA,
            pltpu.SemaphoreType.DMA,
            pltpu.SemaphoreType.DMA,
            pltpu.SemaphoreType.DMA,
        ],
    )
    def k(e2_hbm, dst_hbm, z_hbm, hhp_hbm, di_v, rows_v, hh_sh, sem_i0,
          sem_i1, sem_r0, sem_r1, sem_s0, sem_s1):
        cid = lax.axis_index("c")
        sid = lax.axis_index("s")
        wid = sid * 2 + cid
        base = wid * EPW

        r0 = sid * RPT
        pltpu.sync_copy(z_hbm.at[pl.ds(r0, RPT)], hh_sh.at[pl.ds(r0, RPT)])

        @pl.when(sid == NS - 1)
        def _():
            pltpu.sync_copy(z_hbm.at[pl.ds(NS * RPT, REM)],
                            hh_sh.at[pl.ds(NS * RPT, REM)])

        plsc.subcore_barrier()

        sems = ((sem_i0, sem_r0, sem_s0), (sem_i1, sem_r1, sem_s1))

        def sub(c, p):
            di, rows = di_v.at[p], rows_v.at[p]
            sem_i, sem_r, sem_s = sems[p]
            off = base + c * KB
            gof = goff + off

            @pl.when(c > 1)
            def _():
                for j in range(SUB):
                    pltpu.make_async_copy(rows.at[pl.ds(j * K, K)],
                                          hh_sh.at[di.at[j]], sem_s).wait()

            for j in range(SUB):
                pltpu.async_copy(dst_hbm.at[pl.ds(gof + j * K, K)],
                                 di.at[j], sem_i)
            pltpu.async_copy(e2_hbm.at[pl.ds(off, KB)], rows, sem_r)
            for j in range(SUB):
                pltpu.make_async_copy(dst_hbm.at[pl.ds(gof + j * K, K)],
                                      di.at[j], sem_i).wait()
            pltpu.make_async_copy(e2_hbm.at[pl.ds(off, KB)], rows,
                                  sem_r).wait()
            for j in range(SUB):
                pltpu.async_copy(rows.at[pl.ds(j * K, K)],
                                 hh_sh.at[di.at[j]], sem_s, add=True)

        def step(c, _):
            @pl.when(c % 2 == 0)
            def _():
                sub(c, 0)

            @pl.when(c % 2 == 1)
            def _():
                sub(c, 1)

            return ()

        lax.fori_loop(0, chunks, step, ())
        for cl in (chunks - 2, chunks - 1):
            p = cl % 2
            for j in range(SUB):
                pltpu.make_async_copy(rows_v.at[p].at[pl.ds(j * K, K)],
                                      hh_sh.at[di_v.at[p].at[j]],
                                      sems[p][2]).wait()
        plsc.subcore_barrier()

        pltpu.sync_copy(hh_sh.at[pl.ds(r0, RPT)],
                        hhp_hbm.at[cid, pl.ds(r0, RPT)])

        @pl.when(sid == NS - 1)
        def _():
            pltpu.sync_copy(hh_sh.at[pl.ds(NS * RPT, REM)],
                            hhp_hbm.at[cid, pl.ds(NS * RPT, REM)])

    return k(e2, dst, zeros128)


def kernel(h, e, s, edge_index, params):
    N = h.shape[0]
    E = e.shape[0]
    src = edge_index[0].astype(jnp.int32)
    dst = edge_index[1].astype(jnp.int32)

    row = lambda b: b.reshape(1, -1)
    (Wnf1, bnf1), (Wnf2, bnf2) = params['node_func']
    (Wef1, bef1), (Wef2, bef2) = params['edge_func']
    (Wsf1, bsf1), (Wsf2, bsf2) = params['state_func']
    (Wn1, bn1), (Wn2, bn2), (Wn3, bn3) = params['node_update_func']
    (W1, b1), (W2, b2), (W3, b3) = params['edge_update_func']
    (Ws1, bs1), (Ws2, bs2), (Ws3, bs3) = params['state_update_func']

    W1_hs, W1_hd, W1_e, W1_s = W1[0:128], W1[128:256], W1[256:384], W1[384:512]
    Wn_hh, Wn_h, Wn_s = Wn1[0:128], Wn1[128:256], Wn1[256:384]
    Ws_s1, Ws_ue, Ws_un = Ws1[0:128], Ws1[128:256], Ws1[256:384]

    zeros128 = jnp.zeros((N, 128), F32)

    s8 = jnp.broadcast_to(s, (8, 128))
    s1_8, ce, cn = _state_pre(s8, Wsf1, row(bsf1), Wsf2, row(bsf2),
                              W1_s, row(b1), Wn_s, row(bn1))

    h1 = _node_pre(h, Wnf1, row(bnf1), Wnf2, row(bnf2), bn=2000)

    E2 = E // 2
    e2_halves, accs, hhps = [], [], []
    for i in range(2):
        hs, hd = _sc_gather(h1, src, dst, i * E2, E2)
        e2_i, acc_i = _edge_stage(e, hs, hd, Wef1, row(bef1), Wef2,
                                  row(bef2), W1_hs, W1_hd, W1_e, ce,
                                  W2, row(b2), W3, row(b3), be=2000,
                                  eoff=i * E2)
        hhps.append(_sc_scatter(e2_i, dst, i * E2, zeros128))
        e2_halves.append(e2_i)
        accs.append(acc_i)

    degp = _sc_deg(dst, jnp.ones((80, 128), F32), zeros128)

    e2 = jnp.concatenate(e2_halves, axis=0)
    sum_e2 = accs[0] + accs[1]

    h2, sum_h2 = _node_stage(hhps[0], hhps[1], degp, h1, Wn_hh, Wn_h, cn,
                             Wn2, row(bn2), Wn3, row(bn3), bn=2000)

    s2 = _state_stage(s1_8, sum_e2, sum_h2, E, N, Ws_s1, Ws_ue, Ws_un,
                      row(bs1), Ws2, row(bs2), Ws3, row(bs3))

    return (h2, e2, s2)


# speedup vs baseline: 1.0626x; 1.0626x over previous
import functools
import math

import jax
import jax.numpy as jnp
from jax import lax
from jax.experimental import pallas as pl
from jax.experimental.pallas import tpu as pltpu
from jax.experimental.pallas import tpu_sc as plsc

LN2 = math.log(2.0)
F32 = jnp.float32


def _sp2(x):
    return jnp.log(0.5 + 0.5 * jnp.exp(x))


def _dot(a, b):
    return jnp.dot(a, b, preferred_element_type=F32)


def _state_pre(s8, Wsf1, bsf1, Wsf2, bsf2, W1_s, b1, Wn_s, bn1):
    def body(s_ref, w1_ref, b1_ref, w2_ref, b2_ref, we_ref, be_ref, wn_ref,
             bn_ref, s1_ref, ce_ref, cn_ref):
        t = _sp2(_dot(s_ref[...], w1_ref[...]) + b1_ref[...])
        s1 = _sp2(_dot(t, w2_ref[...]) + b2_ref[...])
        s1_ref[...] = s1
        ce_ref[...] = _dot(s1, we_ref[...]) + be_ref[...]
        cn_ref[...] = _dot(s1, wn_ref[...]) + bn_ref[...]

    return pl.pallas_call(
        body,
        out_shape=(
            jax.ShapeDtypeStruct((8, 128), F32),
            jax.ShapeDtypeStruct((8, 256), F32),
            jax.ShapeDtypeStruct((8, 256), F32),
        ),
    )(s8, Wsf1, bsf1, Wsf2, bsf2, W1_s, b1, Wn_s, bn1)


def _node_pre(h, Wnf1, bnf1, Wnf2, bnf2, bn):
    n = h.shape[0]
    grid = n // bn

    def body(h_ref, w1_ref, b1_ref, w2_ref, b2_ref, h1_ref):
        t = _sp2(_dot(h_ref[...], w1_ref[...]) + b1_ref[...])
        h1_ref[...] = _sp2(_dot(t, w2_ref[...]) + b2_ref[...])

    wspec = lambda shape: pl.BlockSpec(shape, lambda i: (0, 0))
    return pl.pallas_call(
        body,
        grid=(grid,),
        in_specs=[
            pl.BlockSpec((bn, 128), lambda i: (i, 0)),
            wspec((128, 256)), wspec((1, 256)),
            wspec((256, 128)), wspec((1, 128)),
        ],
        out_specs=pl.BlockSpec((bn, 128), lambda i: (i, 0)),
        out_shape=jax.ShapeDtypeStruct((n, 128), F32),
    )(h, Wnf1, bnf1, Wnf2, bnf2)


def _edge_stage(e, hs, hd, Wef1, bef1, Wef2, bef2, W1_hs, W1_hd, W1_e, ce,
                W2, b2, W3, b3, be, eoff):
    cnt = hs.shape[0]
    grid = cnt // be
    ob = eoff // be

    def body(e_ref, hs_ref, hd_ref, wef1_ref, bef1_ref, wef2_ref, bef2_ref,
             whs_ref, whd_ref, we_ref, ce_ref, w2_ref, b2_ref, w3_ref, b3_ref,
             e2_ref, acc_ref):
        t = _sp2(_dot(e_ref[...], wef1_ref[...]) + bef1_ref[...])
        e1 = _sp2(_dot(t, wef2_ref[...]) + bef2_ref[...])
        x = (_dot(hs_ref[...], whs_ref[...]) + _dot(hd_ref[...], whd_ref[...])
             + _dot(e1, we_ref[...]) + ce_ref[0:1, :])
        x = _sp2(x)
        x = _sp2(_dot(x, w2_ref[...]) + b2_ref[...])
        e2 = _sp2(_dot(x, w3_ref[...]) + b3_ref[...])
        e2_ref[...] = e2
        part = jnp.sum(e2.reshape(be // 8, 8, 128), axis=0)

        @pl.when(pl.program_id(0) == 0)
        def _():
            acc_ref[...] = jnp.zeros_like(acc_ref)

        acc_ref[...] += part

    wspec = lambda shape: pl.BlockSpec(shape, lambda i: (0, 0))
    espec = lambda w: pl.BlockSpec((be, w), lambda i: (i, 0))
    return pl.pallas_call(
        body,
        grid=(grid,),
        in_specs=[
            pl.BlockSpec((be, 128), lambda i: (i + ob, 0)),
            espec(128), espec(128),
            wspec((128, 256)), wspec((1, 256)), wspec((256, 128)),
            wspec((1, 128)),
            wspec((128, 256)), wspec((128, 256)), wspec((128, 256)),
            wspec((8, 256)),
            wspec((256, 256)), wspec((1, 256)), wspec((256, 128)),
            wspec((1, 128)),
        ],
        out_specs=(
            pl.BlockSpec((be, 128), lambda i: (i, 0)),
            pl.BlockSpec((8, 128), lambda i: (0, 0)),
        ),
        out_shape=(
            jax.ShapeDtypeStruct((cnt, 128), F32),
            jax.ShapeDtypeStruct((8, 128), F32),
        ),
    )(e, hs, hd, Wef1, bef1, Wef2, bef2, W1_hs, W1_hd, W1_e, ce, W2, b2,
      W3, b3)


def _node_stage(hhp0, hhp1, degp, h1, Wn_hh, Wn_h, cn, Wn2, bn2, Wn3, bn3,
                bn):
    n = h1.shape[0]
    grid = n // bn

    def body(hhp0_ref, hhp1_ref, degp_ref, h1_ref, whh_ref, wh_ref, cn_ref,
             w2_ref, b2_ref, w3_ref, b3_ref, h2_ref, acc_ref):
        hh_sum = (hhp0_ref[0] + hhp0_ref[1]) + (hhp1_ref[0] + hhp1_ref[1])
        deg = degp_ref[0, :, 0:1] + degp_ref[1, :, 0:1]
        hh = hh_sum / jnp.maximum(deg, 1.0)
        x = (_dot(hh, whh_ref[...]) + _dot(h1_ref[...], wh_ref[...])
             + cn_ref[0:1, :])
        x = _sp2(x)
        x = _sp2(_dot(x, w2_ref[...]) + b2_ref[...])
        h2 = _sp2(_dot(x, w3_ref[...]) + b3_ref[...])
        h2_ref[...] = h2
        part = jnp.sum(h2.reshape(bn // 8, 8, 128), axis=0)

        @pl.when(pl.program_id(0) == 0)
        def _():
            acc_ref[...] = jnp.zeros_like(acc_ref)

        acc_ref[...] += part

    wspec = lambda shape: pl.BlockSpec(shape, lambda i: (0, 0))
    return pl.pallas_call(
        body,
        grid=(grid,),
        in_specs=[
            pl.BlockSpec((2, bn, 128), lambda i: (0, i, 0)),
            pl.BlockSpec((2, bn, 128), lambda i: (0, i, 0)),
            pl.BlockSpec((2, bn, 128), lambda i: (0, i, 0)),
            pl.BlockSpec((bn, 128), lambda i: (i, 0)),
            wspec((128, 256)), wspec((128, 256)), wspec((8, 256)),
            wspec((256, 256)), wspec((1, 256)), wspec((256, 128)),
            wspec((1, 128)),
        ],
        out_specs=(
            pl.BlockSpec((bn, 128), lambda i: (i, 0)),
            pl.BlockSpec((8, 128), lambda i: (0, 0)),
        ),
        out_shape=(
            jax.ShapeDtypeStruct((n, 128), F32),
            jax.ShapeDtypeStruct((8, 128), F32),
        ),
    )(hhp0, hhp1, degp, h1, Wn_hh, Wn_h, cn, Wn2, bn2, Wn3, bn3)


def _state_stage(s1_8, sum_e2, sum_h2, n_edges, n_nodes, Ws_s1, Ws_ue, Ws_un,
                 bs1, Ws2, bs2, Ws3, bs3):
    inv_e = 1.0 / float(n_edges)
    inv_n = 1.0 / float(n_nodes)

    def body(s1_ref, se_ref, sh_ref, wa_ref, wb_ref, wc_ref, b1_ref, w2_ref,
             b2_ref, w3_ref, b3_ref, out_ref):
        ue = jnp.sum(se_ref[...], axis=0, keepdims=True) * inv_e
        un = jnp.sum(sh_ref[...], axis=0, keepdims=True) * inv_n
        x = (_dot(s1_ref[0:1, :], wa_ref[...]) + _dot(ue, wb_ref[...])
             + _dot(un, wc_ref[...]) + b1_ref[...])
        x = _sp2(x)
        x = _sp2(_dot(x, w2_ref[...]) + b2_ref[...])
        out_ref[...] = _sp2(_dot(x, w3_ref[...]) + b3_ref[...])

    return pl.pallas_call(
        body,
        out_shape=jax.ShapeDtypeStruct((1, 128), F32),
    )(s1_8, sum_e2, sum_h2, Ws_s1, Ws_ue, Ws_un, bs1, Ws2, bs2, Ws3, bs3)


def _sc_gather(h1, src, dst, goff, cnt):
    NW = 32
    EPW = cnt // NW
    K = 40
    SUB = 5
    KB = K * SUB
    chunks = EPW // KB
    mesh = plsc.VectorSubcoreMesh(core_axis_name="c", subcore_axis_name="s")

    @functools.partial(
        pl.kernel,
        out_type=(
            jax.ShapeDtypeStruct((cnt, 128), F32),
            jax.ShapeDtypeStruct((cnt, 128), F32),
        ),
        mesh=mesh,
        scratch_types=[
            pltpu.VMEM((2, SUB, K), jnp.int32),
            pltpu.VMEM((2, SUB, K), jnp.int32),
            pltpu.VMEM((2, KB, 128), F32),
            pltpu.VMEM((2, KB, 128), F32),
            pltpu.SemaphoreType.DMA,
            pltpu.SemaphoreType.DMA,
            pltpu.SemaphoreType.DMA,
            pltpu.SemaphoreType.DMA,
            pltpu.SemaphoreType.DMA,
            pltpu.SemaphoreType.DMA,
        ],
    )
    def k(h1_hbm, src_hbm, dst_hbm, hs_hbm, hd_hbm,
          si_v, di_v, a_v, b_v, sem_i0, sem_i1, sem_g0, sem_g1, sem_w0,
          sem_w1):
        cid = lax.axis_index("c")
        sid = lax.axis_index("s")
        wid = sid * 2 + cid
        base = wid * EPW
        sems = ((sem_i0, sem_g0, sem_w0), (sem_i1, sem_g1, sem_w1))

        def sub(c, p):
            si, di, av, bv = si_v.at[p], di_v.at[p], a_v.at[p], b_v.at[p]
            sem_i, sem_g, sem_w = sems[p]
            off = base + c * KB
            gof = goff + off
            for j in range(SUB):
                pltpu.async_copy(src_hbm.at[pl.ds(gof + j * K, K)],
                                 si.at[j], sem_i)
                pltpu.async_copy(dst_hbm.at[pl.ds(gof + j * K, K)],
                                 di.at[j], sem_i)

            @pl.when(c > 1)
            def _():
                pltpu.make_async_copy(
                    av, hs_hbm.at[pl.ds(off - 2 * KB, KB)], sem_w).wait()
                pltpu.make_async_copy(
                    bv, hd_hbm.at[pl.ds(off - 2 * KB, KB)], sem_w).wait()

            for j in range(SUB):
                pltpu.make_async_copy(src_hbm.at[pl.ds(gof + j * K, K)],
                                      si.at[j], sem_i).wait()
                pltpu.make_async_copy(dst_hbm.at[pl.ds(gof + j * K, K)],
                                      di.at[j], sem_i).wait()
            for j in range(SUB):
                pltpu.async_copy(h1_hbm.at[si.at[j]],
                                 av.at[pl.ds(j * K, K)], sem_g)
                pltpu.async_copy(h1_hbm.at[di.at[j]],
                                 bv.at[pl.ds(j * K, K)], sem_g)
            for j in range(SUB):
                pltpu.make_async_copy(h1_hbm.at[si.at[j]],
                                      av.at[pl.ds(j * K, K)], sem_g).wait()
                pltpu.make_async_copy(h1_hbm.at[di.at[j]],
                                      bv.at[pl.ds(j * K, K)], sem_g).wait()
            pltpu.async_copy(av, hs_hbm.at[pl.ds(off, KB)], sem_w)
            pltpu.async_copy(bv, hd_hbm.at[pl.ds(off, KB)], sem_w)

        def step(c, _):
            @pl.when(c % 2 == 0)
            def _():
                sub(c, 0)

            @pl.when(c % 2 == 1)
            def _():
                sub(c, 1)

            return ()

        lax.fori_loop(0, chunks, step, ())
        for cl in (chunks - 2, chunks - 1):
            p = cl % 2
            off = base + cl * KB
            pltpu.make_async_copy(a_v.at[p], hs_hbm.at[pl.ds(off, KB)],
                                  sems[p][2]).wait()
            pltpu.make_async_copy(b_v.at[p], hd_hbm.at[pl.ds(off, KB)],
                                  sems[p][2]).wait()

    return k(h1, src, dst)


def _sc_deg(dst, ones80, zeros128):
    E = dst.shape[0]
    N = zeros128.shape[0]
    NW = 32
    NS = 16
    EPW = E // NW
    K = 80
    SUB = 5
    KB = K * SUB
    chunks = EPW // KB
    RPT = 624
    REM = N - NS * RPT
    mesh = plsc.VectorSubcoreMesh(core_axis_name="c", subcore_axis_name="s")

    @functools.partial(
        pl.kernel,
        out_type=jax.ShapeDtypeStruct((2, N, 128), F32),
        mesh=mesh,
        scratch_types=[
            pltpu.VMEM((SUB, K), jnp.int32),
            pltpu.VMEM((K, 128), F32),
            pltpu.VMEM_SHARED((N, 128), F32),
            pltpu.SemaphoreType.DMA,
        ],
    )
    def k(dst_hbm, ones_hbm, z_hbm, deg_hbm, di_v, ones_v, deg_sh, sem_i):
        cid = lax.axis_index("c")
        sid = lax.axis_index("s")
        wid = sid * 2 + cid
        base = wid * EPW

        r0 = sid * RPT
        pltpu.sync_copy(z_hbm.at[pl.ds(r0, RPT)], deg_sh.at[pl.ds(r0, RPT)])

        @pl.when(sid == NS - 1)
        def _():
            pltpu.sync_copy(z_hbm.at[pl.ds(NS * RPT, REM)],
                            deg_sh.at[pl.ds(NS * RPT, REM)])

        pltpu.sync_copy(ones_hbm, ones_v)
        plsc.subcore_barrier()

        def step(c, _):
            off = base + c * KB
            for j in range(SUB):
                pltpu.async_copy(dst_hbm.at[pl.ds(off + j * K, K)],
                                 di_v.at[j], sem_i)
            for j in range(SUB):
                pltpu.make_async_copy(dst_hbm.at[pl.ds(off + j * K, K)],
                                      di_v.at[j], sem_i).wait()
            for j in range(SUB):
                pltpu.sync_copy(ones_v, deg_sh.at[di_v.at[j]], add=True)
            return ()

        lax.fori_loop(0, chunks, step, ())
        plsc.subcore_barrier()

        pltpu.sync_copy(deg_sh.at[pl.ds(r0, RPT)],
                        deg_hbm.at[cid, pl.ds(r0, RPT)])

        @pl.when(sid == NS - 1)
        def _():
            pltpu.sync_copy(deg_sh.at[pl.ds(NS * RPT, REM)],
                            deg_hbm.at[cid, pl.ds(NS * RPT, REM)])

    return k(dst, ones80, zeros128)


def _sc_scatter(e2, dst, goff, zeros128):
    cnt = e2.shape[0]
    N = zeros128.shape[0]
    NW = 32
    NS = 16
    EPW = cnt // NW
    K = 40
    RPT = 624
    REM = N - NS * RPT
    mesh = plsc.VectorSubcoreMesh(core_axis_name="c", subcore_axis_name="s")

    SUB = 5
    KB = K * SUB
    chunks = EPW // KB

    @functools.partial(
        pl.kernel,
        out_type=jax.ShapeDtypeStruct((2, N, 128), F32),
        mesh=mesh,
        scratch_types=[
            pltpu.VMEM((SUB, K), jnp.int32),
            pltpu.VMEM((KB, 128), F32),
            pltpu.VMEM_SHARED((N, 128), F32),
            pltpu.SemaphoreType.DMA,
            pltpu.SemaphoreType.DMA,
        ],
    )
    def k(e2_hbm, dst_hbm, z_hbm, hhp_hbm, di_v, rows_v, hh_sh, sem_i,
          sem_r):
        cid = lax.axis_index("c")
        sid = lax.axis_index("s")
        wid = sid * 2 + cid
        base = wid * EPW

        r0 = sid * RPT
        pltpu.sync_copy(z_hbm.at[pl.ds(r0, RPT)], hh_sh.at[pl.ds(r0, RPT)])

        @pl.when(sid == NS - 1)
        def _():
            pltpu.sync_copy(z_hbm.at[pl.ds(NS * RPT, REM)],
                            hh_sh.at[pl.ds(NS * RPT, REM)])

        plsc.subcore_barrier()

        def step(c, _):
            off = base + c * KB
            gof = goff + off
            for j in range(SUB):
                pltpu.async_copy(dst_hbm.at[pl.ds(gof + j * K, K)],
                                 di_v.at[j], sem_i)
            pltpu.async_copy(e2_hbm.at[pl.ds(off, KB)], rows_v, sem_r)
            for j in range(SUB):
                pltpu.make_async_copy(dst_hbm.at[pl.ds(gof + j * K, K)],
                                      di_v.at[j], sem_i).wait()
            pltpu.make_async_copy(e2_hbm.at[pl.ds(off, KB)], rows_v,
                                  sem_r).wait()
            for j in range(SUB):
                pltpu.sync_copy(rows_v.at[pl.ds(j * K, K)],
                                hh_sh.at[di_v.at[j]], add=True)
            return ()

        lax.fori_loop(0, chunks, step, ())
        plsc.subcore_barrier()

        pltpu.sync_copy(hh_sh.at[pl.ds(r0, RPT)],
                        hhp_hbm.at[cid, pl.ds(r0, RPT)])

        @pl.when(sid == NS - 1)
        def _():
            pltpu.sync_copy(hh_sh.at[pl.ds(NS * RPT, REM)],
                            hhp_hbm.at[cid, pl.ds(NS * RPT, REM)])

    return k(e2, dst, zeros128)


def kernel(h, e, s, edge_index, params):
    N = h.shape[0]
    E = e.shape[0]
    src = edge_index[0].astype(jnp.int32)
    dst = edge_index[1].astype(jnp.int32)

    row = lambda b: b.reshape(1, -1)
    (Wnf1, bnf1), (Wnf2, bnf2) = params['node_func']
    (Wef1, bef1), (Wef2, bef2) = params['edge_func']
    (Wsf1, bsf1), (Wsf2, bsf2) = params['state_func']
    (Wn1, bn1), (Wn2, bn2), (Wn3, bn3) = params['node_update_func']
    (W1, b1), (W2, b2), (W3, b3) = params['edge_update_func']
    (Ws1, bs1), (Ws2, bs2), (Ws3, bs3) = params['state_update_func']

    W1_hs, W1_hd, W1_e, W1_s = W1[0:128], W1[128:256], W1[256:384], W1[384:512]
    Wn_hh, Wn_h, Wn_s = Wn1[0:128], Wn1[128:256], Wn1[256:384]
    Ws_s1, Ws_ue, Ws_un = Ws1[0:128], Ws1[128:256], Ws1[256:384]

    zeros128 = jnp.zeros((N, 128), F32)

    s8 = jnp.broadcast_to(s, (8, 128))
    s1_8, ce, cn = _state_pre(s8, Wsf1, row(bsf1), Wsf2, row(bsf2),
                              W1_s, row(b1), Wn_s, row(bn1))

    h1 = _node_pre(h, Wnf1, row(bnf1), Wnf2, row(bnf2), bn=2000)

    E2 = E // 2
    e2_halves, accs, hhps = [], [], []
    for i in range(2):
        hs, hd = _sc_gather(h1, src, dst, i * E2, E2)
        e2_i, acc_i = _edge_stage(e, hs, hd, Wef1, row(bef1), Wef2,
                                  row(bef2), W1_hs, W1_hd, W1_e, ce,
                                  W2, row(b2), W3, row(b3), be=2000,
                                  eoff=i * E2)
        hhps.append(_sc_scatter(e2_i, dst, i * E2, zeros128))
        e2_halves.append(e2_i)
        accs.append(acc_i)

    degp = _sc_deg(dst, jnp.ones((80, 128), F32), zeros128)

    e2 = jnp.concatenate(e2_halves, axis=0)
    sum_e2 = accs[0] + accs[1]

    h2, sum_h2 = _node_stage(hhps[0], hhps[1], degp, h1, Wn_hh, Wn_h, cn,
                             Wn2, row(bn2), Wn3, row(bn3), bn=2000)

    s2 = _state_stage(s1_8, sum_e2, sum_h2, E, N, Ws_s1, Ws_ue, Ws_un,
                      row(bs1), Ws2, row(bs2), Ws3, row(bs3))

    return (h2, e2, s2)


# speedup vs baseline: 1.1159x; 1.0502x over previous
import functools
import math

import jax
import jax.numpy as jnp
from jax import lax
from jax.experimental import pallas as pl
from jax.experimental.pallas import tpu as pltpu
from jax.experimental.pallas import tpu_sc as plsc

LN2 = math.log(2.0)
F32 = jnp.float32


def _sp2(x):
    return jnp.log(0.5 + 0.5 * jnp.exp(x))


def _dot(a, b):
    return jnp.dot(a, b, preferred_element_type=F32)


def _state_pre(s8, Wsf1, bsf1, Wsf2, bsf2, W1_s, b1, Wn_s, bn1):
    def body(s_ref, w1_ref, b1_ref, w2_ref, b2_ref, we_ref, be_ref, wn_ref,
             bn_ref, s1_ref, ce_ref, cn_ref):
        t = _sp2(_dot(s_ref[...], w1_ref[...]) + b1_ref[...])
        s1 = _sp2(_dot(t, w2_ref[...]) + b2_ref[...])
        s1_ref[...] = s1
        ce_ref[...] = _dot(s1, we_ref[...]) + be_ref[...]
        cn_ref[...] = _dot(s1, wn_ref[...]) + bn_ref[...]

    return pl.pallas_call(
        body,
        out_shape=(
            jax.ShapeDtypeStruct((8, 128), F32),
            jax.ShapeDtypeStruct((8, 256), F32),
            jax.ShapeDtypeStruct((8, 256), F32),
        ),
    )(s8, Wsf1, bsf1, Wsf2, bsf2, W1_s, b1, Wn_s, bn1)


def _node_pre(h, Wnf1, bnf1, Wnf2, bnf2, bn):
    n = h.shape[0]
    grid = n // bn

    def body(h_ref, w1_ref, b1_ref, w2_ref, b2_ref, h1_ref):
        t = _sp2(_dot(h_ref[...], w1_ref[...]) + b1_ref[...])
        h1_ref[...] = _sp2(_dot(t, w2_ref[...]) + b2_ref[...])

    wspec = lambda shape: pl.BlockSpec(shape, lambda i: (0, 0))
    return pl.pallas_call(
        body,
        grid=(grid,),
        in_specs=[
            pl.BlockSpec((bn, 128), lambda i: (i, 0)),
            wspec((128, 256)), wspec((1, 256)),
            wspec((256, 128)), wspec((1, 128)),
        ],
        out_specs=pl.BlockSpec((bn, 128), lambda i: (i, 0)),
        out_shape=jax.ShapeDtypeStruct((n, 128), F32),
    )(h, Wnf1, bnf1, Wnf2, bnf2)


def _edge_stage(e, hs, hd, Wef1, bef1, Wef2, bef2, W1_hs, W1_hd, W1_e, ce,
                W2, b2, W3, b3, be, eoff):
    cnt = hs.shape[0]
    grid = cnt // be
    ob = eoff // be

    def body(e_ref, hs_ref, hd_ref, wef1_ref, bef1_ref, wef2_ref, bef2_ref,
             whs_ref, whd_ref, we_ref, ce_ref, w2_ref, b2_ref, w3_ref, b3_ref,
             e2_ref, acc_ref):
        t = _sp2(_dot(e_ref[...], wef1_ref[...]) + bef1_ref[...])
        e1 = _sp2(_dot(t, wef2_ref[...]) + bef2_ref[...])
        x = (_dot(hs_ref[...], whs_ref[...]) + _dot(hd_ref[...], whd_ref[...])
             + _dot(e1, we_ref[...]) + ce_ref[0:1, :])
        x = _sp2(x)
        x = _sp2(_dot(x, w2_ref[...]) + b2_ref[...])
        e2 = _sp2(_dot(x, w3_ref[...]) + b3_ref[...])
        e2_ref[...] = e2
        part = jnp.sum(e2.reshape(be // 8, 8, 128), axis=0)

        @pl.when(pl.program_id(0) == 0)
        def _():
            acc_ref[...] = jnp.zeros_like(acc_ref)

        acc_ref[...] += part

    wspec = lambda shape: pl.BlockSpec(shape, lambda i: (0, 0))
    espec = lambda w: pl.BlockSpec((be, w), lambda i: (i, 0))
    return pl.pallas_call(
        body,
        grid=(grid,),
        in_specs=[
            pl.BlockSpec((be, 128), lambda i: (i + ob, 0)),
            espec(128), espec(128),
            wspec((128, 256)), wspec((1, 256)), wspec((256, 128)),
            wspec((1, 128)),
            wspec((128, 256)), wspec((128, 256)), wspec((128, 256)),
            wspec((8, 256)),
            wspec((256, 256)), wspec((1, 256)), wspec((256, 128)),
            wspec((1, 128)),
        ],
        out_specs=(
            pl.BlockSpec((be, 128), lambda i: (i, 0)),
            pl.BlockSpec((8, 128), lambda i: (0, 0)),
        ),
        out_shape=(
            jax.ShapeDtypeStruct((cnt, 128), F32),
            jax.ShapeDtypeStruct((8, 128), F32),
        ),
    )(e, hs, hd, Wef1, bef1, Wef2, bef2, W1_hs, W1_hd, W1_e, ce, W2, b2,
      W3, b3)


def _node_stage(hhp0, hhp1, degp, h1, Wn_hh, Wn_h, cn, Wn2, bn2, Wn3, bn3,
                bn):
    n = h1.shape[0]
    grid = n // bn

    def body(hhp0_ref, hhp1_ref, degp_ref, h1_ref, whh_ref, wh_ref, cn_ref,
             w2_ref, b2_ref, w3_ref, b3_ref, h2_ref, acc_ref):
        hh_sum = (hhp0_ref[0] + hhp0_ref[1]) + (hhp1_ref[0] + hhp1_ref[1])
        deg = degp_ref[0, :, 0:1] + degp_ref[1, :, 0:1]
        hh = hh_sum / jnp.maximum(deg, 1.0)
        x = (_dot(hh, whh_ref[...]) + _dot(h1_ref[...], wh_ref[...])
             + cn_ref[0:1, :])
        x = _sp2(x)
        x = _sp2(_dot(x, w2_ref[...]) + b2_ref[...])
        h2 = _sp2(_dot(x, w3_ref[...]) + b3_ref[...])
        h2_ref[...] = h2
        part = jnp.sum(h2.reshape(bn // 8, 8, 128), axis=0)

        @pl.when(pl.program_id(0) == 0)
        def _():
            acc_ref[...] = jnp.zeros_like(acc_ref)

        acc_ref[...] += part

    wspec = lambda shape: pl.BlockSpec(shape, lambda i: (0, 0))
    return pl.pallas_call(
        body,
        grid=(grid,),
        in_specs=[
            pl.BlockSpec((2, bn, 128), lambda i: (0, i, 0)),
            pl.BlockSpec((2, bn, 128), lambda i: (0, i, 0)),
            pl.BlockSpec((2, bn, 128), lambda i: (0, i, 0)),
            pl.BlockSpec((bn, 128), lambda i: (i, 0)),
            wspec((128, 256)), wspec((128, 256)), wspec((8, 256)),
            wspec((256, 256)), wspec((1, 256)), wspec((256, 128)),
            wspec((1, 128)),
        ],
        out_specs=(
            pl.BlockSpec((bn, 128), lambda i: (i, 0)),
            pl.BlockSpec((8, 128), lambda i: (0, 0)),
        ),
        out_shape=(
            jax.ShapeDtypeStruct((n, 128), F32),
            jax.ShapeDtypeStruct((8, 128), F32),
        ),
    )(hhp0, hhp1, degp, h1, Wn_hh, Wn_h, cn, Wn2, bn2, Wn3, bn3)


def _state_stage(s1_8, sum_e2, sum_h2, n_edges, n_nodes, Ws_s1, Ws_ue, Ws_un,
                 bs1, Ws2, bs2, Ws3, bs3):
    inv_e = 1.0 / float(n_edges)
    inv_n = 1.0 / float(n_nodes)

    def body(s1_ref, se_ref, sh_ref, wa_ref, wb_ref, wc_ref, b1_ref, w2_ref,
             b2_ref, w3_ref, b3_ref, out_ref):
        ue = jnp.sum(se_ref[...], axis=0, keepdims=True) * inv_e
        un = jnp.sum(sh_ref[...], axis=0, keepdims=True) * inv_n
        x = (_dot(s1_ref[0:1, :], wa_ref[...]) + _dot(ue, wb_ref[...])
             + _dot(un, wc_ref[...]) + b1_ref[...])
        x = _sp2(x)
        x = _sp2(_dot(x, w2_ref[...]) + b2_ref[...])
        out_ref[...] = _sp2(_dot(x, w3_ref[...]) + b3_ref[...])

    return pl.pallas_call(
        body,
        out_shape=jax.ShapeDtypeStruct((1, 128), F32),
    )(s1_8, sum_e2, sum_h2, Ws_s1, Ws_ue, Ws_un, bs1, Ws2, bs2, Ws3, bs3)


def _sc_gather(h1, src, dst, goff, cnt):
    NW = 32
    EPW = cnt // NW
    K = 40
    SUB = 5
    KB = K * SUB
    chunks = EPW // KB
    mesh = plsc.VectorSubcoreMesh(core_axis_name="c", subcore_axis_name="s")

    @functools.partial(
        pl.kernel,
        out_type=(
            jax.ShapeDtypeStruct((cnt, 128), F32),
            jax.ShapeDtypeStruct((cnt, 128), F32),
        ),
        mesh=mesh,
        scratch_types=[
            pltpu.VMEM((2, SUB, K), jnp.int32),
            pltpu.VMEM((2, SUB, K), jnp.int32),
            pltpu.VMEM((2, KB, 128), F32),
            pltpu.VMEM((2, KB, 128), F32),
            pltpu.SemaphoreType.DMA,
            pltpu.SemaphoreType.DMA,
            pltpu.SemaphoreType.DMA,
            pltpu.SemaphoreType.DMA,
            pltpu.SemaphoreType.DMA,
            pltpu.SemaphoreType.DMA,
        ],
    )
    def k(h1_hbm, src_hbm, dst_hbm, hs_hbm, hd_hbm,
          si_v, di_v, a_v, b_v, sem_i0, sem_i1, sem_g0, sem_g1, sem_w0,
          sem_w1):
        cid = lax.axis_index("c")
        sid = lax.axis_index("s")
        wid = sid * 2 + cid
        base = wid * EPW
        sems = ((sem_i0, sem_g0, sem_w0), (sem_i1, sem_g1, sem_w1))

        def sub(c, p):
            si, di, av, bv = si_v.at[p], di_v.at[p], a_v.at[p], b_v.at[p]
            sem_i, sem_g, sem_w = sems[p]
            off = base + c * KB
            gof = goff + off
            for j in range(SUB):
                pltpu.async_copy(src_hbm.at[pl.ds(gof + j * K, K)],
                                 si.at[j], sem_i)
                pltpu.async_copy(dst_hbm.at[pl.ds(gof + j * K, K)],
                                 di.at[j], sem_i)

            @pl.when(c > 1)
            def _():
                pltpu.make_async_copy(
                    av, hs_hbm.at[pl.ds(off - 2 * KB, KB)], sem_w).wait()
                pltpu.make_async_copy(
                    bv, hd_hbm.at[pl.ds(off - 2 * KB, KB)], sem_w).wait()

            for j in range(SUB):
                pltpu.make_async_copy(src_hbm.at[pl.ds(gof + j * K, K)],
                                      si.at[j], sem_i).wait()
                pltpu.make_async_copy(dst_hbm.at[pl.ds(gof + j * K, K)],
                                      di.at[j], sem_i).wait()
            for j in range(SUB):
                pltpu.async_copy(h1_hbm.at[si.at[j]],
                                 av.at[pl.ds(j * K, K)], sem_g)
                pltpu.async_copy(h1_hbm.at[di.at[j]],
                                 bv.at[pl.ds(j * K, K)], sem_g)
            for j in range(SUB):
                pltpu.make_async_copy(h1_hbm.at[si.at[j]],
                                      av.at[pl.ds(j * K, K)], sem_g).wait()
                pltpu.make_async_copy(h1_hbm.at[di.at[j]],
                                      bv.at[pl.ds(j * K, K)], sem_g).wait()
            pltpu.async_copy(av, hs_hbm.at[pl.ds(off, KB)], sem_w)
            pltpu.async_copy(bv, hd_hbm.at[pl.ds(off, KB)], sem_w)

        def step(c, _):
            @pl.when(c % 2 == 0)
            def _():
                sub(c, 0)

            @pl.when(c % 2 == 1)
            def _():
                sub(c, 1)

            return ()

        lax.fori_loop(0, chunks, step, ())
        for cl in (chunks - 2, chunks - 1):
            p = cl % 2
            off = base + cl * KB
            pltpu.make_async_copy(a_v.at[p], hs_hbm.at[pl.ds(off, KB)],
                                  sems[p][2]).wait()
            pltpu.make_async_copy(b_v.at[p], hd_hbm.at[pl.ds(off, KB)],
                                  sems[p][2]).wait()

    return k(h1, src, dst)


def _sc_deg(dst, ones80, zeros128):
    E = dst.shape[0]
    N = zeros128.shape[0]
    NW = 32
    NS = 16
    EPW = E // NW
    K = 80
    SUB = 5
    KB = K * SUB
    chunks = EPW // KB
    RPT = 624
    REM = N - NS * RPT
    mesh = plsc.VectorSubcoreMesh(core_axis_name="c", subcore_axis_name="s")

    @functools.partial(
        pl.kernel,
        out_type=jax.ShapeDtypeStruct((2, N, 128), F32),
        mesh=mesh,
        scratch_types=[
            pltpu.VMEM((SUB, K), jnp.int32),
            pltpu.VMEM((K, 128), F32),
            pltpu.VMEM_SHARED((N, 128), F32),
            pltpu.SemaphoreType.DMA,
        ],
    )
    def k(dst_hbm, ones_hbm, z_hbm, deg_hbm, di_v, ones_v, deg_sh, sem_i):
        cid = lax.axis_index("c")
        sid = lax.axis_index("s")
        wid = sid * 2 + cid
        base = wid * EPW

        r0 = sid * RPT
        pltpu.sync_copy(z_hbm.at[pl.ds(r0, RPT)], deg_sh.at[pl.ds(r0, RPT)])

        @pl.when(sid == NS - 1)
        def _():
            pltpu.sync_copy(z_hbm.at[pl.ds(NS * RPT, REM)],
                            deg_sh.at[pl.ds(NS * RPT, REM)])

        pltpu.sync_copy(ones_hbm, ones_v)
        plsc.subcore_barrier()

        def step(c, _):
            off = base + c * KB
            for j in range(SUB):
                pltpu.async_copy(dst_hbm.at[pl.ds(off + j * K, K)],
                                 di_v.at[j], sem_i)
            for j in range(SUB):
                pltpu.make_async_copy(dst_hbm.at[pl.ds(off + j * K, K)],
                                      di_v.at[j], sem_i).wait()
            for j in range(SUB):
                pltpu.sync_copy(ones_v, deg_sh.at[di_v.at[j]], add=True)
            return ()

        lax.fori_loop(0, chunks, step, ())
        plsc.subcore_barrier()

        pltpu.sync_copy(deg_sh.at[pl.ds(r0, RPT)],
                        deg_hbm.at[cid, pl.ds(r0, RPT)])

        @pl.when(sid == NS - 1)
        def _():
            pltpu.sync_copy(deg_sh.at[pl.ds(NS * RPT, REM)],
                            deg_hbm.at[cid, pl.ds(NS * RPT, REM)])

    return k(dst, ones80, zeros128)


def _sc_scatter(e2, dst, goff, zeros128):
    cnt = e2.shape[0]
    N = zeros128.shape[0]
    NW = 32
    NS = 16
    EPW = cnt // NW
    K = 40
    RPT = 624
    REM = N - NS * RPT
    mesh = plsc.VectorSubcoreMesh(core_axis_name="c", subcore_axis_name="s")

    SUB = 5
    KB = K * SUB
    chunks = EPW // KB

    @functools.partial(
        pl.kernel,
        out_type=jax.ShapeDtypeStruct((2, N, 128), F32),
        mesh=mesh,
        scratch_types=[
            pltpu.VMEM((SUB, K), jnp.int32),
            pltpu.VMEM((KB, 128), F32),
            pltpu.VMEM_SHARED((N, 128), F32),
            pltpu.SemaphoreType.DMA,
            pltpu.SemaphoreType.DMA,
        ],
    )
    def k(e2_hbm, dst_hbm, z_hbm, hhp_hbm, di_v, rows_v, hh_sh, sem_i,
          sem_r):
        cid = lax.axis_index("c")
        sid = lax.axis_index("s")
        wid = sid * 2 + cid
        base = wid * EPW

        r0 = sid * RPT
        pltpu.sync_copy(z_hbm.at[pl.ds(r0, RPT)], hh_sh.at[pl.ds(r0, RPT)])

        @pl.when(sid == NS - 1)
        def _():
            pltpu.sync_copy(z_hbm.at[pl.ds(NS * RPT, REM)],
                            hh_sh.at[pl.ds(NS * RPT, REM)])

        plsc.subcore_barrier()

        def step(c, _):
            off = base + c * KB
            gof = goff + off
            for j in range(SUB):
                pltpu.async_copy(dst_hbm.at[pl.ds(gof + j * K, K)],
                                 di_v.at[j], sem_i)
            pltpu.async_copy(e2_hbm.at[pl.ds(off, KB)], rows_v, sem_r)
            for j in range(SUB):
                pltpu.make_async_copy(dst_hbm.at[pl.ds(gof + j * K, K)],
                                      di_v.at[j], sem_i).wait()
            pltpu.make_async_copy(e2_hbm.at[pl.ds(off, KB)], rows_v,
                                  sem_r).wait()
            for j in range(SUB):
                pltpu.sync_copy(rows_v.at[pl.ds(j * K, K)],
                                hh_sh.at[di_v.at[j]], add=True)
            return ()

        lax.fori_loop(0, chunks, step, ())
        plsc.subcore_barrier()

        pltpu.sync_copy(hh_sh.at[pl.ds(r0, RPT)],
                        hhp_hbm.at[cid, pl.ds(r0, RPT)])

        @pl.when(sid == NS - 1)
        def _():
            pltpu.sync_copy(hh_sh.at[pl.ds(NS * RPT, REM)],
                            hhp_hbm.at[cid, pl.ds(NS * RPT, REM)])

    return k(e2, dst, zeros128)


def kernel(h, e, s, edge_index, params):
    N = h.shape[0]
    E = e.shape[0]
    src = edge_index[0].astype(jnp.int32)
    dst = edge_index[1].astype(jnp.int32)

    row = lambda b: b.reshape(1, -1)
    (Wnf1, bnf1), (Wnf2, bnf2) = params['node_func']
    (Wef1, bef1), (Wef2, bef2) = params['edge_func']
    (Wsf1, bsf1), (Wsf2, bsf2) = params['state_func']
    (Wn1, bn1), (Wn2, bn2), (Wn3, bn3) = params['node_update_func']
    (W1, b1), (W2, b2), (W3, b3) = params['edge_update_func']
    (Ws1, bs1), (Ws2, bs2), (Ws3, bs3) = params['state_update_func']

    W1_hs, W1_hd, W1_e, W1_s = W1[0:128], W1[128:256], W1[256:384], W1[384:512]
    Wn_hh, Wn_h, Wn_s = Wn1[0:128], Wn1[128:256], Wn1[256:384]
    Ws_s1, Ws_ue, Ws_un = Ws1[0:128], Ws1[128:256], Ws1[256:384]

    zeros128 = jnp.zeros((N, 128), F32)

    s8 = jnp.broadcast_to(s, (8, 128))
    s1_8, ce, cn = _state_pre(s8, Wsf1, row(bsf1), Wsf2, row(bsf2),
                              W1_s, row(b1), Wn_s, row(bn1))

    h1 = _node_pre(h, Wnf1, row(bnf1), Wnf2, row(bnf2), bn=2000)

    E2 = E // 2
    e2_halves, accs, hhps = [], [], []
    for i in range(2):
        hs, hd = _sc_gather(h1, src, dst, i * E2, E2)
        e2_i, acc_i = _edge_stage(e, hs, hd, Wef1, row(bef1), Wef2,
                                  row(bef2), W1_hs, W1_hd, W1_e, ce,
                                  W2, row(b2), W3, row(b3), be=4000,
                                  eoff=i * E2)
        hhps.append(_sc_scatter(e2_i, dst, i * E2, zeros128))
        e2_halves.append(e2_i)
        accs.append(acc_i)

    degp = _sc_deg(dst, jnp.ones((80, 128), F32), zeros128)

    e2 = jnp.concatenate(e2_halves, axis=0)
    sum_e2 = accs[0] + accs[1]

    h2, sum_h2 = _node_stage(hhps[0], hhps[1], degp, h1, Wn_hh, Wn_h, cn,
                             Wn2, row(bn2), Wn3, row(bn3), bn=2000)

    s2 = _state_stage(s1_8, sum_e2, sum_h2, E, N, Ws_s1, Ws_ue, Ws_un,
                      row(bs1), Ws2, row(bs2), Ws3, row(bs3))

    return (h2, e2, s2)
